# baseline (speedup 1.0000x reference)
import jax
import jax.numpy as jnp
from jax import lax
from jax.experimental import pallas as pl
from jax.experimental.pallas import tpu as pltpu, tpu_sc as plsc

NT = 16
LANES = 16


def _rup(x, m):
    return (x + m - 1) // m * m


def _bcast16(idx):
    return jnp.full((LANES,), idx, dtype=jnp.int32)


def _inv_sqrt(d):
    i = plsc.bitcast(d, jnp.int32)
    u = plsc.bitcast(jnp.int32(0x5F3759DF) - (i >> 1), jnp.float32)
    for _ in range(4):
        u = u * (1.5 - 0.5 * d * u * u)
    return u


def _inv_cbrt(v):
    i = plsc.bitcast(v, jnp.int32)
    r = plsc.bitcast(jnp.int32(0x548C2B4B) - i // 3, jnp.float32)
    third = jnp.float32(1.0 / 3.0)
    for _ in range(5):
        r = r * (4.0 - v * r * r * r) * third
    return r


def _make_cheb_kernel(n, n_pad, epp, cin, cout, with_volnorm=False, m_pp=0):
    own = n_pad // NT
    ovr = own // LANES
    nc_in = cin * n_pad
    onc = cin * own
    oc = cout * own
    big = n >= 10000
    red_w = 4 * onc if big else nc_in
    stage_cols = onc // 4 if big else onc
    wb_len = _rup(6 * cin * cout + cout, 8)

    mesh = plsc.VectorSubcoreMesh(core_axis_name="c", subcore_axis_name="s")

    scratch = [
        pltpu.VMEM((nc_in,), jnp.float32),
        pltpu.VMEM((nc_in,), jnp.float32),
        pltpu.VMEM((NT, stage_cols), jnp.float32),
        pltpu.VMEM((epp,), jnp.int32),
        pltpu.VMEM((epp,), jnp.int32),
        pltpu.VMEM((epp,), jnp.float32),
        pltpu.VMEM((onc,), jnp.float32),
        pltpu.VMEM((onc,), jnp.float32),
        pltpu.VMEM((onc,), jnp.float32),
        pltpu.VMEM((oc,), jnp.float32),
        pltpu.VMEM((wb_len,), jnp.float32),
        pltpu.VMEM_SHARED((NT, red_w), jnp.float32),
        pltpu.VMEM_SHARED((nc_in,), jnp.float32),
    ]
    def body(x_hbm, ei_hbm, wb_hbm, *rest):
        if with_volnorm:
            mt_hbm = rest[0]
            rest = rest[1:]
        out_hbm = rest[0]
        (v_full, y_full, stage, row_b, col_b, w_b, acc, tx_a, tx_b,
         out_own, wb_v) = rest[1:12]
        red, bc = rest[12:14]

        cid = lax.axis_index("c")
        tid = lax.axis_index("s")
        fo = tid * onc
        ones = jnp.full((LANES,), 1.0, dtype=jnp.float32)
        lane = lax.iota(jnp.int32, LANES)

        def zero_ref(ref, nwords):
            z = jnp.zeros((LANES,), jnp.float32)

            def zb(i, _):
                ref[pl.ds(i * LANES, LANES)] = z
                return 0

            lax.fori_loop(0, nwords // LANES, zb, 0)

        def reduce_rounds(src_len, my_off, my_len, acc_off):
            nrounds = -(-src_len // red_w)
            for r in range(nrounds):
                lo = r * red_w
                ln = min(red_w, src_len - lo)
                pltpu.sync_copy(y_full.at[pl.ds(lo, ln)],
                                red.at[tid, pl.ds(0, ln)])
                plsc.subcore_barrier()

                @pl.when((my_off >= lo) & (my_off < lo + ln))
                def _():
                    nh = -(-my_len // stage_cols)
                    for h in range(nh):
                        cl = min(stage_cols, my_len - h * stage_cols)
                        pltpu.sync_copy(
                            red.at[:, pl.ds(my_off - lo + h * stage_cols, cl)],
                            stage.at[:, pl.ds(0, cl)])

                        def sb(s, _):
                            def jb(j, _):
                                so = pl.ds(acc_off + h * stage_cols
                                           + j * LANES, LANES)
                                acc[so] = acc[so] + stage[s, pl.ds(j * LANES,
                                                                   LANES)]
                                return 0

                            lax.fori_loop(0, cl // LANES, jb, 0)
                            return 0

                        lax.fori_loop(0, NT, sb, 0)
                plsc.subcore_barrier()

        ne = NT * epp
        pltpu.sync_copy(ei_hbm.at[pl.ds(tid * epp, epp)], row_b)
        pltpu.sync_copy(ei_hbm.at[pl.ds(ne + tid * epp, epp)], col_b)
        pltpu.sync_copy(wb_hbm, wb_v)
        pltpu.sync_copy(x_hbm, v_full)

        zero_ref(y_full, n_pad)

        def deg_body(i, _):
            r = row_b[pl.ds(i * LANES, LANES)]
            plsc.addupdate_scatter(y_full, [r], ones)
            return 0

        lax.fori_loop(0, epp // LANES, deg_body, 0)
        zero_ref(acc, own)
        reduce_rounds(n_pad, tid * own, own, 0)

        for j in range(ovr):
            sl = pl.ds(j * LANES, LANES)
            d = acc[sl]
            acc[sl] = jnp.where(d > 0.5, _inv_sqrt(jnp.maximum(d, 0.5)), 0.0)
        pltpu.sync_copy(acc.at[pl.ds(0, own)], bc.at[pl.ds(tid * own, own)])
        plsc.subcore_barrier()
        pltpu.sync_copy(bc.at[pl.ds(0, n_pad)], y_full.at[pl.ds(0, n_pad)])

        def wbody(i, _):
            sl = pl.ds(i * LANES, LANES)
            dr = plsc.load_gather(y_full, [row_b[sl]])
            dc = plsc.load_gather(y_full, [col_b[sl]])
            w_b[sl] = -(dr * dc)
            return 0

        lax.fori_loop(0, epp // LANES, wbody, 0)

        boff = 6 * cin * cout
        zero_ref(out_own, oc)

        def bias_body(co, _):
            bv = plsc.load_gather(wb_v, [_bcast16(boff + co)])
            for j in range(ovr):
                plsc.addupdate_scatter(
                    out_own, [lane * cout + (j * LANES * cout + co)], bv)
            return 0

        lax.fori_loop(0, cout, bias_body, 0)

        def mm_accum(k, src, src_off):
            def mm_body(q, _):
                ci = q // cout
                co = q - ci * cout
                wv = plsc.load_gather(
                    wb_v, [_bcast16((k * cin + ci) * cout + co)])
                for j in range(ovr):
                    xv = plsc.load_gather(
                        src, [lane * cin + (src_off + j * LANES * cin + ci)])
                    plsc.addupdate_scatter(
                        out_own, [lane * cout + (j * LANES * cout + co)],
                        xv * wv)
                return 0

            lax.fori_loop(0, cin * cout, mm_body, 0)

        mm_accum(0, v_full, fo)
        tx_prev, tx_curr = tx_a, tx_b

        for k in range(1, 6):
            zero_ref(y_full, nc_in)

            def prop_body(i, _):
                sl = pl.ds(i * LANES, LANES)
                r = row_b[sl] * cin
                c = col_b[sl] * cin
                wv = w_b[sl]
                for ch in range(cin):
                    vals = plsc.load_gather(v_full, [c + ch]) * wv
                    plsc.addupdate_scatter(y_full, [r + ch], vals)
                return 0

            lax.fori_loop(0, epp // LANES, prop_body, 0)

            for j in range(onc // LANES):
                sl = pl.ds(j * LANES, LANES)
                tx_curr[sl] = v_full[pl.ds(fo + j * LANES, LANES)]

            zero_ref(acc, onc)
            reduce_rounds(nc_in, fo, onc, 0)

            if k > 1:
                for j in range(onc // LANES):
                    sl = pl.ds(j * LANES, LANES)
                    acc[sl] = 2.0 * acc[sl] - tx_prev[sl]
            mm_accum(k, acc, 0)

            pltpu.sync_copy(acc.at[pl.ds(0, onc)], bc.at[pl.ds(fo, onc)])
            plsc.subcore_barrier()
            pltpu.sync_copy(bc, v_full)

            tx_prev, tx_curr = tx_curr, tx_prev

        if not with_volnorm:
            @pl.when(cid == 0)
            def _():
                pltpu.sync_copy(out_own, out_hbm.at[pl.ds(tid * oc, oc)])
        else:
            nm = NT * m_pp
            pltpu.sync_copy(mt_hbm.at[pl.ds(tid * m_pp, m_pp)],
                            row_b.at[pl.ds(0, m_pp)])
            pltpu.sync_copy(mt_hbm.at[pl.ds(nm + tid * m_pp, m_pp)],
                            row_b.at[pl.ds(m_pp, m_pp)])
            pltpu.sync_copy(mt_hbm.at[pl.ds(2 * nm + tid * m_pp, m_pp)],
                            col_b.at[pl.ds(0, m_pp)])
            pltpu.sync_copy(out_own, bc.at[pl.ds(tid * oc, oc)])
            plsc.subcore_barrier()
            pltpu.sync_copy(bc, v_full)

            def tri_body(i, part):
                sl = pl.ds(i * LANES, LANES)
                ia = row_b[sl] * 3
                ib = row_b[pl.ds(m_pp + i * LANES, LANES)] * 3
                ic = col_b[sl] * 3
                a0 = plsc.load_gather(v_full, [ia])
                a1 = plsc.load_gather(v_full, [ia + 1])
                a2 = plsc.load_gather(v_full, [ia + 2])
                b0 = plsc.load_gather(v_full, [ib])
                b1 = plsc.load_gather(v_full, [ib + 1])
                b2 = plsc.load_gather(v_full, [ib + 2])
                c0 = plsc.load_gather(v_full, [ic])
                c1 = plsc.load_gather(v_full, [ic + 1])
                c2 = plsc.load_gather(v_full, [ic + 2])
                det = (a0 * (b1 * c2 - b2 * c1)
                       - a1 * (b0 * c2 - b2 * c0)
                       + a2 * (b0 * c1 - b1 * c0))
                return part + jnp.abs(det)

            part = lax.fori_loop(0, m_pp // LANES, tri_body,
                                 jnp.zeros((LANES,), jnp.float32))
            acc[pl.ds(0, LANES)] = part
            pltpu.sync_copy(acc.at[pl.ds(0, LANES)],
                            red.at[tid, pl.ds(0, LANES)])
            plsc.subcore_barrier()

            def sum_body(s, tot):
                pltpu.sync_copy(red.at[s, pl.ds(0, LANES)],
                                acc.at[pl.ds(0, LANES)])
                return tot + acc[pl.ds(0, LANES)]

            tot = lax.fori_loop(0, NT, sum_body,
                                jnp.zeros((LANES,), jnp.float32))
            vol = jnp.sum(tot, axis=0) * jnp.float32(1.0 / 6.0)
            rscale = _inv_cbrt(jnp.full((LANES,), vol, jnp.float32))

            for j in range(oc // LANES):
                sl = pl.ds(j * LANES, LANES)
                out_own[sl] = v_full[pl.ds(tid * oc + j * LANES, LANES)] * rscale

            @pl.when(cid == 0)
            def _():
                pltpu.sync_copy(out_own, out_hbm.at[pl.ds(tid * oc, oc)])

    return pl.kernel(
        body,
        out_type=jax.ShapeDtypeStruct((cout * n_pad,), jnp.float32),
        mesh=mesh,
        scratch_types=scratch,
        compiler_params=pltpu.CompilerParams(
            needs_layout_passes=False, use_tc_tiling_on_sc=False),
        name=f"sc_cheb_n{n}",
    )


def _tc_upsample(s_mat, t, n_pad_out, block_n):
    nc, nf = s_mat.shape
    c = t.shape[1]
    grid = (nf + block_n - 1) // block_n

    def body(t_ref, s_ref, o_ref):
        y = lax.dot_general(s_ref[...], t_ref[...],
                            (((0,), (0,)), ((), ())),
                            preferred_element_type=jnp.float32)
        o_ref[...] = jnp.where(y > 0, y, jnp.exp(y) - 1.0)

    out = pl.pallas_call(
        body,
        grid=(grid,),
        in_specs=[
            pl.BlockSpec((nc, c), lambda i: (0, 0)),
            pl.BlockSpec((nc, block_n), lambda i: (0, i)),
        ],
        out_specs=pl.BlockSpec((block_n, c), lambda i: (i, 0)),
        out_shape=jax.ShapeDtypeStruct((nf, c), jnp.float32),
    )(t, s_mat)
    return jnp.pad(out, ((0, n_pad_out - nf), (0, 0)))


def _pad_edges(ei, n, epp):
    e = ei.shape[1]
    ei = ei.astype(jnp.int32)
    pad = NT * epp - e
    if pad:
        ei = jnp.concatenate(
            [ei, jnp.full((2, pad), n, dtype=jnp.int32)], axis=1)
    return ei.reshape(-1)


def _pack_wb(w, b):
    flat = jnp.concatenate([w.reshape(-1), b.reshape(-1)])
    return jnp.pad(flat, (0, _rup(flat.shape[0], 8) - flat.shape[0]))


_LEVELS = [
    (320, 5120, 1, 16, 512),
    (625, 10000, 16, 8, 768),
    (1250, 20000, 8, 4, 1280),
    (2500, 40000, 4, 2, 2560),
    (5000, 80000, 2, 3, 5120),
    (10000, 160000, 3, 3, 10240),
]

_M_PP = _rup(20000 // NT, LANES)

_CHEB = []
for _i, (_n, _e, _ci, _co, _np_) in enumerate(_LEVELS):
    _epp = _rup(_e // NT, LANES)
    _CHEB.append(_make_cheb_kernel(
        _n, _np_, _epp, _ci, _co,
        with_volnorm=(_i == 5), m_pp=_M_PP if _i == 5 else 0))


def kernel(z, edge_index_0, edge_index_1, edge_index_2, edge_index_3,
           edge_index_4, edge_index_5, S0, S1, S2, S3, S4, M,
           W1, b1, W2, b2, W3, b3, W4, b4, W5, b5, W6, b6):
    edges = [edge_index_5, edge_index_4, edge_index_3, edge_index_2,
             edge_index_1, edge_index_0]
    smats = [S4, S3, S2, S1, S0]
    ws = [(W1, b1), (W2, b2), (W3, b3), (W4, b4), (W5, b5), (W6, b6)]
    blocks = [625, 1250, 2500, 1024, 1024]

    mt = M.astype(jnp.int32).T
    mt = jnp.pad(mt, ((0, 0), (0, NT * _M_PP - mt.shape[1]))).reshape(-1)

    x = jnp.pad(z.astype(jnp.float32), ((0, 512 - 320), (0, 0)))
    for i, (n, e, ci, co, n_pad) in enumerate(_LEVELS):
        epp = _rup(e // NT, LANES)
        ei = _pad_edges(edges[i], n, epp)
        wb = _pack_wb(*ws[i])
        if i < 5:
            x = _CHEB[i](x.reshape(-1), ei, wb).reshape(n_pad, co)
            nxt_pad = _LEVELS[i + 1][4]
            x = _tc_upsample(smats[i], x[:n], nxt_pad, blocks[i])
        else:
            x = _CHEB[i](x.reshape(-1), ei, wb, mt).reshape(n_pad, co)
    return x[:10000]


# speedup vs baseline: 1.4115x; 1.4115x over previous
import jax
import jax.numpy as jnp
from jax import lax
from jax.experimental import pallas as pl
from jax.experimental.pallas import tpu as pltpu, tpu_sc as plsc

NT = 16
LANES = 16


def _rup(x, m):
    return (x + m - 1) // m * m


def _bcast16(idx):
    return jnp.full((LANES,), idx, dtype=jnp.int32)


def _inv_sqrt(d):
    i = plsc.bitcast(d, jnp.int32)
    u = plsc.bitcast(jnp.int32(0x5F3759DF) - (i >> 1), jnp.float32)
    for _ in range(4):
        u = u * (1.5 - 0.5 * d * u * u)
    return u


def _inv_cbrt(v):
    i = plsc.bitcast(v, jnp.int32)
    r = plsc.bitcast(jnp.int32(0x548C2B4B) - i // 3, jnp.float32)
    third = jnp.float32(1.0 / 3.0)
    for _ in range(5):
        r = r * (4.0 - v * r * r * r) * third
    return r


def _make_cheb_kernel(n, n_pad, epp, cin, cout, with_volnorm=False, m_pp=0):
    own = n_pad // NT
    ovr = own // LANES
    nc_in = cin * n_pad
    onc = cin * own
    oc = cout * own
    big = n >= 10000
    red_w = nc_in // 2 if big else nc_in
    stage_cols = 480 if big else min(onc, 960)
    wb_len = _rup(6 * cin * cout + cout, 8)

    mesh = plsc.VectorSubcoreMesh(core_axis_name="c", subcore_axis_name="s")

    scratch = [
        pltpu.VMEM((nc_in,), jnp.float32),
        pltpu.VMEM((nc_in,), jnp.float32),
        pltpu.VMEM((NT, stage_cols), jnp.float32),
        pltpu.VMEM((epp,), jnp.int32),
        pltpu.VMEM((epp,), jnp.float32),
        pltpu.VMEM((onc,), jnp.float32),
        pltpu.VMEM((onc,), jnp.float32),
        pltpu.VMEM((onc,), jnp.float32),
        pltpu.VMEM((oc,), jnp.float32),
        pltpu.VMEM((wb_len,), jnp.float32),
        pltpu.VMEM_SHARED((NT, red_w), jnp.float32),
        pltpu.VMEM_SHARED((nc_in,), jnp.float32),
    ]
    def body(x_hbm, ei_hbm, wb_hbm, *rest):
        if with_volnorm:
            mt_hbm = rest[0]
            rest = rest[1:]
        out_hbm = rest[0]
        (v_full, y_full, stage, pk_b, w_b, acc, tx_a, tx_b,
         out_own, wb_v) = rest[1:11]
        red, bc = rest[11:13]

        cid = lax.axis_index("c")
        tid = lax.axis_index("s")
        fo = tid * onc
        ones = jnp.full((LANES,), 1.0, dtype=jnp.float32)
        lane = lax.iota(jnp.int32, LANES)

        def zero_ref(ref, nwords):
            z = jnp.zeros((LANES,), jnp.float32)
            nv = nwords // LANES
            un = 8 if nv % 8 == 0 else 1

            def zb(i, _):
                for u in range(un):
                    ref[pl.ds((i * un + u) * LANES, LANES)] = z
                return 0

            lax.fori_loop(0, nv // un, zb, 0)

        def col_sum(src_len, dst, dst_off):
            nrounds = -(-src_len // red_w)
            for r in range(nrounds):
                lo = r * red_w
                ln = min(red_w, src_len - lo)
                blk = ln // NT
                if r > 0:
                    plsc.subcore_barrier()
                pltpu.sync_copy(y_full.at[pl.ds(lo, ln)],
                                red.at[tid, pl.ds(0, ln)])
                plsc.subcore_barrier()
                nh = -(-blk // stage_cols)
                for h in range(nh):
                    ho = h * stage_cols
                    cl = min(stage_cols, blk - ho)
                    pltpu.sync_copy(
                        red.at[:, pl.ds(tid * blk + ho, cl)],
                        stage.at[:, pl.ds(0, cl)])
                    for j in range(cl // LANES):
                        acc[pl.ds(ho + j * LANES, LANES)] = (
                            stage[0, pl.ds(j * LANES, LANES)]
                            + stage[1, pl.ds(j * LANES, LANES)])

                    def sb(s, _):
                        for j in range(cl // LANES):
                            ao = pl.ds(ho + j * LANES, LANES)
                            acc[ao] = acc[ao] + stage[s, pl.ds(j * LANES,
                                                               LANES)]
                        return 0

                    lax.fori_loop(2, NT, sb, 0)
                    if dst is None:
                        pltpu.sync_copy(
                            acc.at[pl.ds(ho, cl)],
                            bc.at[pl.ds(lo + tid * blk + ho, cl)])
            return nrounds

        pltpu.sync_copy(ei_hbm.at[pl.ds(tid * epp, epp)], pk_b)
        pltpu.sync_copy(wb_hbm, wb_v)
        pltpu.sync_copy(x_hbm, v_full)

        zero_ref(y_full, n_pad)

        def deg_body(i, _):
            r = pk_b[pl.ds(i * LANES, LANES)] >> 14
            plsc.addupdate_scatter(y_full, [r], ones)
            return 0

        lax.fori_loop(0, epp // LANES, deg_body, 0)
        col_sum(n_pad, acc, 0)

        def dinv_body(j, _):
            sl = pl.ds(j * LANES, LANES)
            d = acc[sl]
            acc[sl] = jnp.where(d > 0.5, _inv_sqrt(jnp.maximum(d, 0.5)), 0.0)
            return 0

        lax.fori_loop(0, ovr, dinv_body, 0)
        pltpu.sync_copy(acc.at[pl.ds(0, own)], bc.at[pl.ds(tid * own, own)])
        plsc.subcore_barrier()
        pltpu.sync_copy(bc.at[pl.ds(0, n_pad)], y_full.at[pl.ds(0, n_pad)])

        def wbody(i, _):
            sl = pl.ds(i * LANES, LANES)
            p = pk_b[sl]
            dr = plsc.load_gather(y_full, [p >> 14])
            dc = plsc.load_gather(y_full, [p & 16383])
            w_b[sl] = -(dr * dc)
            return 0

        lax.fori_loop(0, epp // LANES, wbody, 0)

        boff = 6 * cin * cout
        zero_ref(out_own, oc)

        def bias_body(co, _):
            bv = plsc.load_gather(wb_v, [_bcast16(boff + co)])
            for j in range(ovr):
                plsc.addupdate_scatter(
                    out_own, [lane * cout + (j * LANES * cout + co)], bv)
            return 0

        lax.fori_loop(0, cout, bias_body, 0)

        def mm_accum(k, src, src_off):
            def mm_body(q, _):
                ci = q // cout
                co = q - ci * cout
                wv = plsc.load_gather(
                    wb_v, [_bcast16((k * cin + ci) * cout + co)])

                def mm_j(j, _):
                    xv = plsc.load_gather(
                        src, [lane * cin + (src_off + j * LANES * cin + ci)])
                    plsc.addupdate_scatter(
                        out_own, [lane * cout + (j * LANES * cout + co)],
                        xv * wv)
                    return 0

                lax.fori_loop(0, ovr, mm_j, 0)
                return 0

            lax.fori_loop(0, cin * cout, mm_body, 0)

        mm_accum(0, v_full, fo)
        tx_prev, tx_curr = tx_a, tx_b

        for k in range(1, 6):
            zero_ref(y_full, nc_in)

            def prop_body(i, _):
                sl = pl.ds(i * LANES, LANES)
                p = pk_b[sl]
                r = (p >> 14) * cin
                c = (p & 16383) * cin
                wv = w_b[sl]
                for ch in range(cin):
                    vals = plsc.load_gather(v_full, [c + ch]) * wv
                    plsc.addupdate_scatter(y_full, [r + ch], vals)
                return 0

            lax.fori_loop(0, epp // LANES, prop_body, 0)

            def save_body(j, _):
                sl = pl.ds(j * LANES, LANES)
                tx_curr[sl] = v_full[pl.ds(fo + j * LANES, LANES)]
                return 0

            lax.fori_loop(0, onc // LANES, save_body, 0)

            if red_w >= nc_in:
                col_sum(nc_in, acc, 0)
            else:
                col_sum(nc_in, None, 0)
                plsc.subcore_barrier()
                pltpu.sync_copy(bc.at[pl.ds(fo, onc)], acc)

            if k > 1:
                txp = tx_prev

                def rec_body(j, _):
                    sl = pl.ds(j * LANES, LANES)
                    acc[sl] = 2.0 * acc[sl] - txp[sl]
                    return 0

                lax.fori_loop(0, onc // LANES, rec_body, 0)
            mm_accum(k, acc, 0)

            pltpu.sync_copy(acc.at[pl.ds(0, onc)], bc.at[pl.ds(fo, onc)])
            plsc.subcore_barrier()
            pltpu.sync_copy(bc, v_full)

            tx_prev, tx_curr = tx_curr, tx_prev

        if not with_volnorm:
            @pl.when(cid == 0)
            def _():
                pltpu.sync_copy(out_own, out_hbm.at[pl.ds(tid * oc, oc)])
        else:
            nm = NT * m_pp
            pltpu.sync_copy(mt_hbm.at[pl.ds(tid * m_pp, m_pp)],
                            pk_b.at[pl.ds(0, m_pp)])
            pltpu.sync_copy(mt_hbm.at[pl.ds(nm + tid * m_pp, m_pp)],
                            pk_b.at[pl.ds(m_pp, m_pp)])
            pltpu.sync_copy(mt_hbm.at[pl.ds(2 * nm + tid * m_pp, m_pp)],
                            pk_b.at[pl.ds(2 * m_pp, m_pp)])
            pltpu.sync_copy(out_own, bc.at[pl.ds(tid * oc, oc)])
            plsc.subcore_barrier()
            pltpu.sync_copy(bc, v_full)

            def tri_body(i, part):
                sl = pl.ds(i * LANES, LANES)
                ia = pk_b[sl] * 3
                ib = pk_b[pl.ds(m_pp + i * LANES, LANES)] * 3
                ic = pk_b[pl.ds(2 * m_pp + i * LANES, LANES)] * 3
                a0 = plsc.load_gather(v_full, [ia])
                a1 = plsc.load_gather(v_full, [ia + 1])
                a2 = plsc.load_gather(v_full, [ia + 2])
                b0 = plsc.load_gather(v_full, [ib])
                b1 = plsc.load_gather(v_full, [ib + 1])
                b2 = plsc.load_gather(v_full, [ib + 2])
                c0 = plsc.load_gather(v_full, [ic])
                c1 = plsc.load_gather(v_full, [ic + 1])
                c2 = plsc.load_gather(v_full, [ic + 2])
                det = (a0 * (b1 * c2 - b2 * c1)
                       - a1 * (b0 * c2 - b2 * c0)
                       + a2 * (b0 * c1 - b1 * c0))
                return part + jnp.abs(det)

            part = lax.fori_loop(0, m_pp // LANES, tri_body,
                                 jnp.zeros((LANES,), jnp.float32))
            acc[pl.ds(0, LANES)] = part
            pltpu.sync_copy(acc.at[pl.ds(0, LANES)],
                            red.at[tid, pl.ds(0, LANES)])
            plsc.subcore_barrier()

            def sum_body(s, tot):
                pltpu.sync_copy(red.at[s, pl.ds(0, LANES)],
                                acc.at[pl.ds(0, LANES)])
                return tot + acc[pl.ds(0, LANES)]

            tot = lax.fori_loop(0, NT, sum_body,
                                jnp.zeros((LANES,), jnp.float32))
            vol = jnp.sum(tot, axis=0) * jnp.float32(1.0 / 6.0)
            rscale = _inv_cbrt(jnp.full((LANES,), vol, jnp.float32))

            def scale_body(j, _):
                sl = pl.ds(j * LANES, LANES)
                out_own[sl] = v_full[pl.ds(tid * oc + j * LANES,
                                           LANES)] * rscale
                return 0

            lax.fori_loop(0, oc // LANES, scale_body, 0)

            @pl.when(cid == 0)
            def _():
                pltpu.sync_copy(out_own, out_hbm.at[pl.ds(tid * oc, oc)])

    return pl.kernel(
        body,
        out_type=jax.ShapeDtypeStruct((cout * n_pad,), jnp.float32),
        mesh=mesh,
        scratch_types=scratch,
        compiler_params=pltpu.CompilerParams(
            needs_layout_passes=False, use_tc_tiling_on_sc=False),
        name=f"sc_cheb_n{n}",
    )


def _tc_upsample(s_mat, t, n_pad_out, block_n):
    nc, nf = s_mat.shape
    c = t.shape[1]
    grid = (nf + block_n - 1) // block_n

    def body(t_ref, s_ref, o_ref):
        y = lax.dot_general(s_ref[...], t_ref[...],
                            (((0,), (0,)), ((), ())),
                            preferred_element_type=jnp.float32)
        o_ref[...] = jnp.where(y > 0, y, jnp.exp(y) - 1.0)

    out = pl.pallas_call(
        body,
        grid=(grid,),
        in_specs=[
            pl.BlockSpec((nc, c), lambda i: (0, 0)),
            pl.BlockSpec((nc, block_n), lambda i: (0, i)),
        ],
        out_specs=pl.BlockSpec((block_n, c), lambda i: (i, 0)),
        out_shape=jax.ShapeDtypeStruct((nf, c), jnp.float32),
    )(t, s_mat)
    return jnp.pad(out, ((0, n_pad_out - nf), (0, 0)))


def _pad_edges(ei, n, epp):
    e = ei.shape[1]
    ei = ei.astype(jnp.int32)
    pk = (ei[0] << 14) | ei[1]
    pad = NT * epp - e
    if pad:
        pk = jnp.concatenate(
            [pk, jnp.full((pad,), (n << 14) | n, dtype=jnp.int32)])
    return pk


def _pack_wb(w, b):
    flat = jnp.concatenate([w.reshape(-1), b.reshape(-1)])
    return jnp.pad(flat, (0, _rup(flat.shape[0], 8) - flat.shape[0]))


_LEVELS = [
    (320, 5120, 1, 16, 512),
    (625, 10000, 16, 8, 768),
    (1250, 20000, 8, 4, 1280),
    (2500, 40000, 4, 2, 2560),
    (5000, 80000, 2, 3, 5120),
    (10000, 160000, 3, 3, 10240),
]

_M_PP = _rup(20000 // NT, LANES)

_CHEB = []
for _i, (_n, _e, _ci, _co, _np_) in enumerate(_LEVELS):
    _epp = _rup(_e // NT, LANES)
    _CHEB.append(_make_cheb_kernel(
        _n, _np_, _epp, _ci, _co,
        with_volnorm=(_i == 5), m_pp=_M_PP if _i == 5 else 0))


def kernel(z, edge_index_0, edge_index_1, edge_index_2, edge_index_3,
           edge_index_4, edge_index_5, S0, S1, S2, S3, S4, M,
           W1, b1, W2, b2, W3, b3, W4, b4, W5, b5, W6, b6):
    edges = [edge_index_5, edge_index_4, edge_index_3, edge_index_2,
             edge_index_1, edge_index_0]
    smats = [S4, S3, S2, S1, S0]
    ws = [(W1, b1), (W2, b2), (W3, b3), (W4, b4), (W5, b5), (W6, b6)]
    blocks = [625, 1250, 2500, 1024, 1024]

    mt = M.astype(jnp.int32).T
    mt = jnp.pad(mt, ((0, 0), (0, NT * _M_PP - mt.shape[1]))).reshape(-1)

    x = jnp.pad(z.astype(jnp.float32), ((0, 512 - 320), (0, 0)))
    for i, (n, e, ci, co, n_pad) in enumerate(_LEVELS):
        epp = _rup(e // NT, LANES)
        ei = _pad_edges(edges[i], n, epp)
        wb = _pack_wb(*ws[i])
        if i < 5:
            x = _CHEB[i](x.reshape(-1), ei, wb).reshape(n_pad, co)
            nxt_pad = _LEVELS[i + 1][4]
            x = _tc_upsample(smats[i], x[:n], nxt_pad, blocks[i])
        else:
            x = _CHEB[i](x.reshape(-1), ei, wb, mt).reshape(n_pad, co)
    return x[:10000]
